# Initial kernel scaffold; baseline (speedup 1.0000x reference)
import jax
import jax.numpy as jnp
from jax.experimental import pallas as pl


def kernel(atom_features, bond_features, bond_index, molecule_features, atom_incoming_bond_map, bond_reverse_map, atom_chunk_mask, molecule_chunk_mask, Wi_w, Wi_b, Wm_w, Wm_b, Wa_w, Wa_b, bn_g, bn_b):
    raise NotImplementedError("write your pallas kernel here")


# speedup vs baseline: 5.3896x; 5.3896x over previous
import functools

import jax
import jax.numpy as jnp
from jax import lax
from jax.experimental import pallas as pl
from jax.experimental.pallas import tpu as pltpu
from jax.experimental.pallas import tpu_sc as plsc

N_ATOMS = 10000
N_BONDS = 160000
ATOM_DIM = 128
BOND_DIM = 16
HIDDEN = 128
MAXB = 8
N_MOL = 256
MOL_DIM = 32
DEPTH = 3
EPS = 1e-5
SLOPE = 0.01

NC, NS = 2, 16
NW = NC * NS
BLK = 2000
NB_E = N_BONDS // BLK
E_PAD = N_BONDS + BLK
NB_EP = E_PAD // BLK
A_PAD = 10240
NB_A = N_ATOMS // BLK
A_OUT = N_ATOMS


@functools.cache
def _get_mesh():
    return plsc.VectorSubcoreMesh(
        core_axis_name="c", subcore_axis_name="s", num_cores=NC, num_subcores=NS
    )


def _wid():
    return lax.axis_index("s") * NC + lax.axis_index("c")


def _gather_body(n_per_w, chunk, tab_hbm, idx_hbm, out_hbm,
                 idx0_v, idx1_v, buf0_v, buf1_v, sem0, sem1):
    base0 = _wid() * n_per_w
    n_chunks = n_per_w // chunk
    idx_v = (idx0_v, idx1_v)
    buf_v = (buf0_v, buf1_v)
    sem = (sem0, sem1)

    def start(s, base):
        pltpu.sync_copy(idx_hbm.at[pl.ds(base, chunk)], idx_v[s])
        pltpu.async_copy(tab_hbm.at[idx_v[s]], buf_v[s], sem[s])

    for s in range(min(2, n_chunks)):
        start(s, base0 + s * chunk)

    def slot_step(s, c):
        pltpu.make_async_copy(tab_hbm.at[idx_v[s]], buf_v[s], sem[s]).wait()
        pltpu.sync_copy(buf_v[s], out_hbm.at[pl.ds(base0 + c * chunk, chunk)])

        @pl.when(c + 2 < n_chunks)
        def _():
            start(s, base0 + (c + 2) * chunk)

    def chunk_fn(c, carry):
        @pl.when(c % 2 == 0)
        def _():
            slot_step(0, c)

        @pl.when(c % 2 == 1)
        def _():
            slot_step(1, c)

        return carry

    lax.fori_loop(0, n_chunks, chunk_fn, None)


def _mt_body(n_per_w, chunk, s_hbm, h_hbm, bi0_hbm, brm_hbm, out_hbm,
             i0a_v, i1a_v, i0b_v, i1b_v, b0a_v, b1a_v, b0b_v, b1b_v,
             sem0a, sem1a, sem0b, sem1b):
    base0 = _wid() * n_per_w
    n_chunks = n_per_w // chunk
    ia_v = (i0a_v, i1a_v)
    ib_v = (i0b_v, i1b_v)
    ba_v = (b0a_v, b1a_v)
    bb_v = (b0b_v, b1b_v)
    sa = (sem0a, sem1a)
    sb = (sem0b, sem1b)

    def start(s, base):
        pltpu.sync_copy(bi0_hbm.at[pl.ds(base, chunk)], ia_v[s])
        pltpu.sync_copy(brm_hbm.at[pl.ds(base, chunk)], ib_v[s])
        pltpu.async_copy(s_hbm.at[ia_v[s]], ba_v[s], sa[s])
        pltpu.async_copy(h_hbm.at[ib_v[s]], bb_v[s], sb[s])

    for s in range(min(2, n_chunks)):
        start(s, base0 + s * chunk)

    def slot_step(s, c):
        pltpu.make_async_copy(s_hbm.at[ia_v[s]], ba_v[s], sa[s]).wait()
        pltpu.make_async_copy(h_hbm.at[ib_v[s]], bb_v[s], sb[s]).wait()

        def row(r, c2):
            for j in range(HIDDEN // 16):
                sl = pl.ds(j * 16, 16)
                ba_v[s][r, sl] = ba_v[s][r, sl] - bb_v[s][r, sl]
            return c2

        lax.fori_loop(0, chunk, row, None)
        pltpu.sync_copy(ba_v[s], out_hbm.at[pl.ds(base0 + c * chunk, chunk)])

        @pl.when(c + 2 < n_chunks)
        def _():
            start(s, base0 + (c + 2) * chunk)

    def chunk_fn(c, carry):
        @pl.when(c % 2 == 0)
        def _():
            slot_step(0, c)

        @pl.when(c % 2 == 1)
        def _():
            slot_step(1, c)

        return carry

    lax.fori_loop(0, n_chunks, chunk_fn, None)


def _gather_sum_body(n_per_w, ca, k, tab_hbm, idx_hbm, out_hbm,
                     idx0_v, idx1_v, buf0_v, buf1_v, acc_v, sem0, sem1):
    base0 = _wid() * n_per_w
    n_chunks = n_per_w // ca
    idx_v = (idx0_v, idx1_v)
    buf_v = (buf0_v, buf1_v)
    sem = (sem0, sem1)

    def start(s, arow):
        pltpu.sync_copy(idx_hbm.at[pl.ds(arow * k, ca * k)], idx_v[s])
        pltpu.async_copy(tab_hbm.at[idx_v[s]], buf_v[s], sem[s])

    for s in range(min(2, n_chunks)):
        start(s, base0 + s * ca)

    def slot_step(s, c):
        pltpu.make_async_copy(tab_hbm.at[idx_v[s]], buf_v[s], sem[s]).wait()

        def atom(a, c2):
            for j in range(HIDDEN // 16):
                sl = pl.ds(j * 16, 16)
                acc = buf_v[s][a * k, sl]
                for kk in range(1, k):
                    acc = acc + buf_v[s][a * k + kk, sl]
                acc_v[a, sl] = acc
            return c2

        lax.fori_loop(0, ca, atom, None)
        pltpu.sync_copy(acc_v, out_hbm.at[pl.ds(base0 + c * ca, ca)])

        @pl.when(c + 2 < n_chunks)
        def _():
            start(s, base0 + (c + 2) * ca)

    def chunk_fn(c, carry):
        @pl.when(c % 2 == 0)
        def _():
            slot_step(0, c)

        @pl.when(c % 2 == 1)
        def _():
            slot_step(1, c)

        return carry

    lax.fori_loop(0, n_chunks, chunk_fn, None)


def _sc_gather(tab, idx, n_rows, chunk=200):
    n_per_w = n_rows // NW
    body = functools.partial(_gather_body, n_per_w, chunk)
    return pl.kernel(
        body,
        out_type=jax.ShapeDtypeStruct((n_rows, HIDDEN), jnp.float32),
        mesh=_get_mesh(),
        scratch_types=[
            pltpu.VMEM((chunk,), jnp.int32),
            pltpu.VMEM((chunk,), jnp.int32),
            pltpu.VMEM((chunk, HIDDEN), jnp.float32),
            pltpu.VMEM((chunk, HIDDEN), jnp.float32),
            pltpu.SemaphoreType.DMA,
            pltpu.SemaphoreType.DMA,
        ],
    )(tab, idx)


def _sc_mt(s_tab, h_tab, bi0, brm, chunk=200):
    n_per_w = N_BONDS // NW
    body = functools.partial(_mt_body, n_per_w, chunk)
    return pl.kernel(
        body,
        out_type=jax.ShapeDtypeStruct((N_BONDS, HIDDEN), jnp.float32),
        mesh=_get_mesh(),
        scratch_types=[
            pltpu.VMEM((chunk,), jnp.int32),
            pltpu.VMEM((chunk,), jnp.int32),
            pltpu.VMEM((chunk,), jnp.int32),
            pltpu.VMEM((chunk,), jnp.int32),
            pltpu.VMEM((chunk, HIDDEN), jnp.float32),
            pltpu.VMEM((chunk, HIDDEN), jnp.float32),
            pltpu.VMEM((chunk, HIDDEN), jnp.float32),
            pltpu.VMEM((chunk, HIDDEN), jnp.float32),
            pltpu.SemaphoreType.DMA,
            pltpu.SemaphoreType.DMA,
            pltpu.SemaphoreType.DMA,
            pltpu.SemaphoreType.DMA,
        ],
    )(s_tab, h_tab, bi0, brm)


def _sc_gather_sum(tab, idx_flat, n_out, k):
    ca = (320 // k) // 8 * 8
    n_per_w = n_out // NW
    body = functools.partial(_gather_sum_body, n_per_w, ca, k)
    return pl.kernel(
        body,
        out_type=jax.ShapeDtypeStruct((n_out, HIDDEN), jnp.float32),
        mesh=_get_mesh(),
        scratch_types=[
            pltpu.VMEM((ca * k,), jnp.int32),
            pltpu.VMEM((ca * k,), jnp.int32),
            pltpu.VMEM((ca * k, HIDDEN), jnp.float32),
            pltpu.VMEM((ca * k, HIDDEN), jnp.float32),
            pltpu.VMEM((ca, HIDDEN), jnp.float32),
            pltpu.SemaphoreType.DMA,
            pltpu.SemaphoreType.DMA,
        ],
    )(tab, idx_flat)


def _lrelu(y):
    return jnp.where(y >= 0, y, SLOPE * y)


def _row_mask(i, n_valid):
    rows = i * BLK + lax.broadcasted_iota(jnp.int32, (BLK, HIDDEN), 0)
    return rows < n_valid


def _stats_update(i, z, stats_ref):
    @pl.when(i == 0)
    def _():
        stats_ref[...] = jnp.zeros((8, HIDDEN), jnp.float32)

    zs = jnp.sum(z, axis=0, keepdims=True)
    zq = jnp.sum(z * z, axis=0, keepdims=True)
    stats_ref[0:1, :] = stats_ref[0:1, :] + zs
    stats_ref[1:2, :] = stats_ref[1:2, :] + zq


def _proj_body(x_ref, w_ref, o_ref):
    o_ref[...] = lax.dot_general(
        x_ref[...], w_ref[...], (((1,), (1,)), ((), ())),
        preferred_element_type=jnp.float32)


def _tc_proj(x, w, n_rows, nb):
    return pl.pallas_call(
        _proj_body,
        grid=(nb,),
        in_specs=[
            pl.BlockSpec((BLK, x.shape[1]), lambda i: (i, 0)),
            pl.BlockSpec(w.shape, lambda i: (0, 0)),
        ],
        out_specs=pl.BlockSpec((BLK, HIDDEN), lambda i: (i, 0)),
        out_shape=jax.ShapeDtypeStruct((n_rows, HIDDEN), jnp.float32),
    )(x, w)


def _h0_body(bf_ref, pg_ref, w_ref, b_ref, z_ref, stats_ref):
    i = pl.program_id(0)
    y = lax.dot_general(bf_ref[...], w_ref[...], (((1,), (1,)), ((), ())),
                        preferred_element_type=jnp.float32)
    y = y + pg_ref[...] + b_ref[0:1, :]
    z = jnp.where(_row_mask(i, N_BONDS), _lrelu(y), 0.0)
    z_ref[...] = z
    _stats_update(i, z, stats_ref)


def _tc_h0(bf, pg, w, b8):
    last = NB_E - 1
    return pl.pallas_call(
        _h0_body,
        grid=(NB_EP,),
        in_specs=[
            pl.BlockSpec((BLK, BOND_DIM), lambda i: (jnp.minimum(i, last), 0)),
            pl.BlockSpec((BLK, HIDDEN), lambda i: (jnp.minimum(i, last), 0)),
            pl.BlockSpec((HIDDEN, BOND_DIM), lambda i: (0, 0)),
            pl.BlockSpec((8, HIDDEN), lambda i: (0, 0)),
        ],
        out_specs=[
            pl.BlockSpec((BLK, HIDDEN), lambda i: (i, 0)),
            pl.BlockSpec((8, HIDDEN), lambda i: (0, 0)),
        ],
        out_shape=[
            jax.ShapeDtypeStruct((E_PAD, HIDDEN), jnp.float32),
            jax.ShapeDtypeStruct((8, HIDDEN), jnp.float32),
        ],
    )(bf, pg, w, b8)


def _depth_body(mt_ref, h0_ref, w_ref, b_ref, z_ref, stats_ref):
    i = pl.program_id(0)
    y = lax.dot_general(mt_ref[...], w_ref[...], (((1,), (1,)), ((), ())),
                        preferred_element_type=jnp.float32)
    y = y + h0_ref[...] + b_ref[0:1, :]
    z = jnp.where(_row_mask(i, N_BONDS), _lrelu(y), 0.0)
    z_ref[...] = z
    _stats_update(i, z, stats_ref)


def _tc_depth(mt, h0, w, b8):
    last = NB_E - 1
    return pl.pallas_call(
        _depth_body,
        grid=(NB_EP,),
        in_specs=[
            pl.BlockSpec((BLK, HIDDEN), lambda i: (jnp.minimum(i, last), 0)),
            pl.BlockSpec((BLK, HIDDEN), lambda i: (i, 0)),
            pl.BlockSpec((HIDDEN, HIDDEN), lambda i: (0, 0)),
            pl.BlockSpec((8, HIDDEN), lambda i: (0, 0)),
        ],
        out_specs=[
            pl.BlockSpec((BLK, HIDDEN), lambda i: (i, 0)),
            pl.BlockSpec((8, HIDDEN), lambda i: (0, 0)),
        ],
        out_shape=[
            jax.ShapeDtypeStruct((E_PAD, HIDDEN), jnp.float32),
            jax.ShapeDtypeStruct((8, HIDDEN), jnp.float32),
        ],
    )(mt, h0, w, b8)


def _atom_body(af_ref, sv_ref, wa_ref, wm_ref, b_ref, z_ref, stats_ref):
    i = pl.program_id(0)
    y = lax.dot_general(af_ref[...], wa_ref[...], (((1,), (1,)), ((), ())),
                        preferred_element_type=jnp.float32)
    y = y + lax.dot_general(sv_ref[...], wm_ref[...], (((1,), (1,)), ((), ())),
                            preferred_element_type=jnp.float32)
    y = y + b_ref[0:1, :]
    z = jnp.where(_row_mask(i, N_ATOMS), _lrelu(y), 0.0)
    z_ref[...] = z
    _stats_update(i, z, stats_ref)


def _tc_atom(af, sv, wa, wm, b8):
    return pl.pallas_call(
        _atom_body,
        grid=(NB_A,),
        in_specs=[
            pl.BlockSpec((BLK, ATOM_DIM), lambda i: (i, 0)),
            pl.BlockSpec((BLK, HIDDEN), lambda i: (i, 0)),
            pl.BlockSpec((HIDDEN, ATOM_DIM), lambda i: (0, 0)),
            pl.BlockSpec((HIDDEN, HIDDEN), lambda i: (0, 0)),
            pl.BlockSpec((8, HIDDEN), lambda i: (0, 0)),
        ],
        out_specs=[
            pl.BlockSpec((BLK, HIDDEN), lambda i: (i, 0)),
            pl.BlockSpec((8, HIDDEN), lambda i: (0, 0)),
        ],
        out_shape=[
            jax.ShapeDtypeStruct((A_OUT, HIDDEN), jnp.float32),
            jax.ShapeDtypeStruct((8, HIDDEN), jnp.float32),
        ],
    )(af, sv, wa, wm, b8)


def _norm_body(n_valid, z_ref, sc_ref, sh_ref, o_ref):
    i = pl.program_id(0)
    h = z_ref[...] * sc_ref[0:1, :] + sh_ref[0:1, :]
    o_ref[...] = jnp.where(_row_mask(i, n_valid), h, 0.0)


def _tc_norm(z, scale8, shift8, n_valid):
    n_pad = z.shape[0]
    body = functools.partial(_norm_body, n_valid)
    return pl.pallas_call(
        body,
        grid=(n_pad // BLK,),
        in_specs=[
            pl.BlockSpec((BLK, HIDDEN), lambda i: (i, 0)),
            pl.BlockSpec((8, HIDDEN), lambda i: (0, 0)),
            pl.BlockSpec((8, HIDDEN), lambda i: (0, 0)),
        ],
        out_specs=pl.BlockSpec((BLK, HIDDEN), lambda i: (i, 0)),
        out_shape=jax.ShapeDtypeStruct((n_pad, HIDDEN), jnp.float32),
    )(z, scale8, shift8)


def _bn_coeffs(stats, n, g, b):
    mean = stats[0] / n
    var = stats[1] / n - mean * mean
    scale = g * lax.rsqrt(var + EPS)
    shift = b - mean * scale
    return (jnp.broadcast_to(scale[None, :], (8, HIDDEN)),
            jnp.broadcast_to(shift[None, :], (8, HIDDEN)))


def kernel(atom_features, bond_features, bond_index, molecule_features,
           atom_incoming_bond_map, bond_reverse_map, atom_chunk_mask,
           molecule_chunk_mask, Wi_w, Wi_b, Wm_w, Wm_b, Wa_w, Wa_b,
           bn_g, bn_b):
    bi0 = bond_index[0]
    aibm2 = jnp.where(atom_incoming_bond_map > 0,
                      atom_incoming_bond_map - 1, N_BONDS)
    aibm_flat = jnp.concatenate(
        [aibm2, jnp.full((A_PAD - N_ATOMS, MAXB), N_BONDS, jnp.int32)]
    ).reshape(-1)
    acm_flat = jnp.concatenate(
        [atom_chunk_mask,
         jnp.zeros((A_PAD - N_ATOMS, atom_chunk_mask.shape[1]), jnp.int32)]
    ).reshape(-1)
    mcm_flat = molecule_chunk_mask.reshape(-1)

    Wi_bond = Wi_w[:, :BOND_DIM]
    Wi_atom = Wi_w[:, BOND_DIM:]
    Wa_atom = Wa_w[:, :ATOM_DIM]
    Wa_msg = Wa_w[:, ATOM_DIM:]
    wi_b8 = jnp.broadcast_to(Wi_b[None, :], (8, HIDDEN))
    wm_b8 = jnp.broadcast_to(Wm_b[None, :], (8, HIDDEN))
    wa_b8 = jnp.broadcast_to(Wa_b[None, :], (8, HIDDEN))

    proj = _tc_proj(atom_features, Wi_atom, N_ATOMS, NB_A)
    proj_g = _sc_gather(proj, bi0, N_BONDS)
    z0, st0 = _tc_h0(bond_features, proj_g, Wi_bond, wi_b8)
    sc0, sh0 = _bn_coeffs(st0, float(N_BONDS), bn_g, bn_b)
    h0 = _tc_norm(z0, sc0, sh0, N_BONDS)

    h = h0
    for _ in range(DEPTH):
        s_atom = _sc_gather_sum(h, aibm_flat, A_PAD, MAXB)
        mt = _sc_mt(s_atom, h, bi0, bond_reverse_map)
        z, st = _tc_depth(mt, h0, Wm_w, wm_b8)
        sc, sh = _bn_coeffs(st, float(N_BONDS), bn_g, bn_b)
        h = _tc_norm(z, sc, sh, N_BONDS)

    sv = _sc_gather_sum(h, acm_flat, A_PAD, atom_chunk_mask.shape[1])
    z2, st2 = _tc_atom(atom_features, sv, Wa_atom, Wa_msg, wa_b8)
    sc2, sh2 = _bn_coeffs(st2, float(N_ATOMS), bn_g, bn_b)
    hv = _tc_norm(z2, sc2, sh2, N_ATOMS)

    hm = _sc_gather_sum(hv, mcm_flat, N_MOL, molecule_chunk_mask.shape[1])
    return jnp.concatenate([hm, molecule_features], axis=1)
